# baseline (speedup 1.0000x reference)
import functools

import jax
import jax.numpy as jnp
from jax import lax
from jax.experimental import pallas as pl
from jax.experimental.pallas import tpu as pltpu
from jax.experimental.pallas import tpu_sc as plsc

N_NODES = 10000
N_EDGES = 320000
D_FEAT = 128
D_HALF = 64
D_DEG = 16

NUM_CORES = 2
NUM_SUBCORES = 16
NUM_WORKERS = NUM_CORES * NUM_SUBCORES
EDGES_PER_WORKER = N_EDGES // NUM_WORKERS
CHUNK = 40
NUM_CHUNKS = EDGES_PER_WORKER // CHUNK
ROWS_PER_TILE = N_NODES // NUM_SUBCORES
ZROWS = 25


def _sc_body(flo_hbm, fhi_hbm, src_hbm, dst_hbm, outa_hbm, outb_hbm, outd_hbm,
             acca_sh, accb_sh, accd_sh, src_v, dst_v,
             a0, a1, b0, b1, ones_v, zerof_v, zerod_v,
             ga0, ga1, gb0, gb1, sa, sd):
    cid = lax.axis_index("c")
    sid = lax.axis_index("s")
    wid = cid * NUM_SUBCORES + sid

    zvec = jnp.zeros((16,), jnp.float32)
    ovec = jnp.ones((16,), jnp.float32)

    def _zrow(i, _):
        for k in range(D_HALF // 16):
            zerof_v[i, pl.ds(k * 16, 16)] = zvec
        zerod_v[i, pl.ds(0, 16)] = zvec
        return 0

    lax.fori_loop(0, ZROWS, _zrow, 0)

    def _orow(i, _):
        ones_v[i, pl.ds(0, 16)] = ovec
        return 0

    lax.fori_loop(0, CHUNK, _orow, 0)

    row0 = sid * ROWS_PER_TILE

    def _zcopy(q, _):
        pltpu.sync_copy(zerof_v, acca_sh.at[pl.ds(row0 + q * ZROWS, ZROWS)])
        pltpu.sync_copy(zerof_v, accb_sh.at[pl.ds(row0 + q * ZROWS, ZROWS)])
        pltpu.sync_copy(zerod_v, accd_sh.at[pl.ds(row0 + q * ZROWS, ZROWS)])
        return 0

    lax.fori_loop(0, ROWS_PER_TILE // ZROWS, _zcopy, 0)

    pltpu.sync_copy(src_hbm.at[wid], src_v)
    pltpu.sync_copy(dst_hbm.at[wid], dst_v)

    plsc.subcore_barrier()

    pltpu.async_copy(flo_hbm.at[src_v.at[0]], a0, ga0)
    pltpu.async_copy(fhi_hbm.at[src_v.at[0]], b0, gb0)
    pltpu.async_copy(flo_hbm.at[src_v.at[1]], a1, ga1)
    pltpu.async_copy(fhi_hbm.at[src_v.at[1]], b1, gb1)

    def _step(i, _):
        j = 2 * i

        pltpu.make_async_copy(flo_hbm.at[src_v.at[j]], a0, ga0).wait()
        pltpu.make_async_copy(fhi_hbm.at[src_v.at[j]], b0, gb0).wait()

        @pl.when(i > 0)
        def _():
            pltpu.make_async_copy(ones_v, accd_sh.at[dst_v.at[0]], sd).wait()

        pltpu.async_copy(ones_v, accd_sh.at[dst_v.at[j]], sd, add=True)
        pltpu.async_copy(a0, acca_sh.at[dst_v.at[j]], sa, add=True)
        pltpu.sync_copy(b0, accb_sh.at[dst_v.at[j]], add=True)
        pltpu.make_async_copy(a0, acca_sh.at[dst_v.at[j]], sa).wait()

        @pl.when(i < NUM_CHUNKS // 2 - 1)
        def _():
            pltpu.async_copy(flo_hbm.at[src_v.at[j + 2]], a0, ga0)
            pltpu.async_copy(fhi_hbm.at[src_v.at[j + 2]], b0, gb0)

        pltpu.make_async_copy(flo_hbm.at[src_v.at[j + 1]], a1, ga1).wait()
        pltpu.make_async_copy(fhi_hbm.at[src_v.at[j + 1]], b1, gb1).wait()
        pltpu.make_async_copy(ones_v, accd_sh.at[dst_v.at[0]], sd).wait()
        pltpu.async_copy(ones_v, accd_sh.at[dst_v.at[j + 1]], sd, add=True)
        pltpu.async_copy(a1, acca_sh.at[dst_v.at[j + 1]], sa, add=True)
        pltpu.sync_copy(b1, accb_sh.at[dst_v.at[j + 1]], add=True)
        pltpu.make_async_copy(a1, acca_sh.at[dst_v.at[j + 1]], sa).wait()

        @pl.when(i < NUM_CHUNKS // 2 - 1)
        def _():
            pltpu.async_copy(flo_hbm.at[src_v.at[j + 3]], a1, ga1)
            pltpu.async_copy(fhi_hbm.at[src_v.at[j + 3]], b1, gb1)

        return 0

    lax.fori_loop(0, NUM_CHUNKS // 2, _step, 0)

    pltpu.make_async_copy(ones_v, accd_sh.at[dst_v.at[0]], sd).wait()

    plsc.subcore_barrier()

    out_base = cid * N_NODES + sid * ROWS_PER_TILE
    pltpu.sync_copy(acca_sh.at[pl.ds(row0, ROWS_PER_TILE)],
                    outa_hbm.at[pl.ds(out_base, ROWS_PER_TILE)])
    pltpu.sync_copy(accb_sh.at[pl.ds(row0, ROWS_PER_TILE)],
                    outb_hbm.at[pl.ds(out_base, ROWS_PER_TILE)])
    pltpu.sync_copy(accd_sh.at[pl.ds(row0, ROWS_PER_TILE)],
                    outd_hbm.at[pl.ds(out_base, ROWS_PER_TILE)])


@functools.lru_cache(maxsize=1)
def _sc_agg():
    return functools.partial(
        pl.kernel,
        out_type=(
            jax.ShapeDtypeStruct((NUM_CORES * N_NODES, D_HALF), jnp.float32),
            jax.ShapeDtypeStruct((NUM_CORES * N_NODES, D_HALF), jnp.float32),
            jax.ShapeDtypeStruct((NUM_CORES * N_NODES, D_DEG), jnp.float32),
        ),
        mesh=plsc.VectorSubcoreMesh(core_axis_name="c", subcore_axis_name="s"),
        scratch_types=[
            pltpu.VMEM_SHARED((N_NODES, D_HALF), jnp.float32),
            pltpu.VMEM_SHARED((N_NODES, D_HALF), jnp.float32),
            pltpu.VMEM_SHARED((N_NODES, D_DEG), jnp.float32),
            pltpu.VMEM((NUM_CHUNKS, CHUNK), jnp.int32),
            pltpu.VMEM((NUM_CHUNKS, CHUNK), jnp.int32),
            pltpu.VMEM((CHUNK, D_HALF), jnp.float32),
            pltpu.VMEM((CHUNK, D_HALF), jnp.float32),
            pltpu.VMEM((CHUNK, D_HALF), jnp.float32),
            pltpu.VMEM((CHUNK, D_HALF), jnp.float32),
            pltpu.VMEM((CHUNK, D_DEG), jnp.float32),
            pltpu.VMEM((ZROWS, D_HALF), jnp.float32),
            pltpu.VMEM((ZROWS, D_DEG), jnp.float32),
            pltpu.SemaphoreType.DMA,
            pltpu.SemaphoreType.DMA,
            pltpu.SemaphoreType.DMA,
            pltpu.SemaphoreType.DMA,
            pltpu.SemaphoreType.DMA,
            pltpu.SemaphoreType.DMA,
        ],
        compiler_params=pltpu.CompilerParams(use_tc_tiling_on_sc=False),
    )(_sc_body)


def _tc_body(pa_ref, pb_ref, pd_ref, f_ref, wt_ref, b_ref, o_ref):
    sum_lo = pa_ref[0] + pa_ref[1]
    sum_hi = pb_ref[0] + pb_ref[1]
    feat_sum = jnp.concatenate([sum_lo, sum_hi], axis=1)
    deg = pd_ref[0] + pd_ref[1]
    mean = feat_sum / jnp.maximum(deg, 1.0)
    h = jnp.where(deg > 0.0, mean, f_ref[...])
    y = jnp.dot(h, wt_ref[...], preferred_element_type=jnp.float32)
    o_ref[...] = jnp.maximum(y + b_ref[...], 0.0)


_BR = 1000


def _tc_finish(pa, pb, pdeg, features, wt, b2):
    grid = (N_NODES // _BR,)
    return pl.pallas_call(
        _tc_body,
        grid=grid,
        in_specs=[
            pl.BlockSpec((NUM_CORES, _BR, D_HALF), lambda i: (0, i, 0)),
            pl.BlockSpec((NUM_CORES, _BR, D_HALF), lambda i: (0, i, 0)),
            pl.BlockSpec((NUM_CORES, _BR, 1), lambda i: (0, i, 0)),
            pl.BlockSpec((_BR, D_FEAT), lambda i: (i, 0)),
            pl.BlockSpec((D_FEAT, D_FEAT), lambda i: (0, 0)),
            pl.BlockSpec((1, D_FEAT), lambda i: (0, 0)),
        ],
        out_specs=pl.BlockSpec((_BR, D_FEAT), lambda i: (i, 0)),
        out_shape=jax.ShapeDtypeStruct((N_NODES, D_FEAT), jnp.float32),
    )(pa, pb, pdeg, features, wt, b2)


def kernel(features, edge_index, W, b):
    src = edge_index[0].astype(jnp.int32).reshape(NUM_WORKERS, NUM_CHUNKS, CHUNK)
    dst = edge_index[1].astype(jnp.int32).reshape(NUM_WORKERS, NUM_CHUNKS, CHUNK)
    flo = features[:, :D_HALF]
    fhi = features[:, D_HALF:]
    pa, pb, pdeg = _sc_agg()(flo, fhi, src, dst)
    pa = pa.reshape(NUM_CORES, N_NODES, D_HALF)
    pb = pb.reshape(NUM_CORES, N_NODES, D_HALF)
    pdeg = pdeg.reshape(NUM_CORES, N_NODES, D_DEG)[:, :, :1]
    return _tc_finish(pa, pb, pdeg, features, W.T, b.reshape(1, D_FEAT))


# speedup vs baseline: 1.0713x; 1.0713x over previous
import functools

import jax
import jax.numpy as jnp
from jax import lax
from jax.experimental import pallas as pl
from jax.experimental.pallas import tpu as pltpu
from jax.experimental.pallas import tpu_sc as plsc

N_NODES = 10000
N_EDGES = 320000
D_FEAT = 128
D_DEG = 16

NUM_CORES = 2
NUM_SUBCORES = 16
NUM_WORKERS = NUM_CORES * NUM_SUBCORES
EDGES_PER_WORKER = N_EDGES // NUM_WORKERS
CHUNK = 40
NUM_CHUNKS = EDGES_PER_WORKER // CHUNK
ROWS_PER_TILE = N_NODES // NUM_SUBCORES


def _sc_body(feat_hbm, src_hbm, dst_hbm, zf_hbm, zd_hbm, outf_hbm, outd_hbm,
             accf_sh, accd_sh, src_v, dst_v, rows0, rows1, ones_v,
             g0, g1, sd, zs):
    cid = lax.axis_index("c")
    sid = lax.axis_index("s")
    wid = cid * NUM_SUBCORES + sid
    row0 = sid * ROWS_PER_TILE

    pltpu.async_copy(zf_hbm, accf_sh.at[pl.ds(row0, ROWS_PER_TILE)], zs)
    pltpu.async_copy(zd_hbm, accd_sh.at[pl.ds(row0, ROWS_PER_TILE)], zs)

    ovec = jnp.ones((16,), jnp.float32)

    def _orow(i, _):
        ones_v[i, pl.ds(0, 16)] = ovec
        return 0

    lax.fori_loop(0, CHUNK, _orow, 0)

    pltpu.sync_copy(src_hbm.at[wid], src_v)
    pltpu.sync_copy(dst_hbm.at[wid], dst_v)

    pltpu.make_async_copy(zf_hbm, accf_sh.at[pl.ds(row0, ROWS_PER_TILE)], zs).wait()
    pltpu.make_async_copy(zd_hbm, accd_sh.at[pl.ds(row0, ROWS_PER_TILE)], zs).wait()

    plsc.subcore_barrier()

    pltpu.async_copy(feat_hbm.at[src_v.at[0]], rows0, g0)
    pltpu.async_copy(feat_hbm.at[src_v.at[1]], rows1, g1)

    def _step(i, _):
        j = 2 * i
        pltpu.make_async_copy(feat_hbm.at[src_v.at[j]], rows0, g0).wait()

        @pl.when(i > 0)
        def _():
            pltpu.make_async_copy(ones_v, accd_sh.at[dst_v.at[0]], sd).wait()

        pltpu.async_copy(ones_v, accd_sh.at[dst_v.at[j]], sd, add=True)
        pltpu.sync_copy(rows0, accf_sh.at[dst_v.at[j]], add=True)

        @pl.when(i < NUM_CHUNKS // 2 - 1)
        def _():
            pltpu.async_copy(feat_hbm.at[src_v.at[j + 2]], rows0, g0)

        pltpu.make_async_copy(feat_hbm.at[src_v.at[j + 1]], rows1, g1).wait()
        pltpu.make_async_copy(ones_v, accd_sh.at[dst_v.at[0]], sd).wait()
        pltpu.async_copy(ones_v, accd_sh.at[dst_v.at[j + 1]], sd, add=True)
        pltpu.sync_copy(rows1, accf_sh.at[dst_v.at[j + 1]], add=True)

        @pl.when(i < NUM_CHUNKS // 2 - 1)
        def _():
            pltpu.async_copy(feat_hbm.at[src_v.at[j + 3]], rows1, g1)

        return 0

    lax.fori_loop(0, NUM_CHUNKS // 2, _step, 0)

    pltpu.make_async_copy(ones_v, accd_sh.at[dst_v.at[0]], sd).wait()

    plsc.subcore_barrier()

    out_base = cid * N_NODES + sid * ROWS_PER_TILE
    pltpu.sync_copy(accf_sh.at[pl.ds(row0, ROWS_PER_TILE)],
                    outf_hbm.at[pl.ds(out_base, ROWS_PER_TILE)])
    pltpu.sync_copy(accd_sh.at[pl.ds(row0, ROWS_PER_TILE)],
                    outd_hbm.at[pl.ds(out_base, ROWS_PER_TILE)])


@functools.lru_cache(maxsize=1)
def _sc_agg():
    return functools.partial(
        pl.kernel,
        out_type=(
            jax.ShapeDtypeStruct((NUM_CORES * N_NODES, D_FEAT), jnp.float32),
            jax.ShapeDtypeStruct((NUM_CORES * N_NODES, D_DEG), jnp.float32),
        ),
        mesh=plsc.VectorSubcoreMesh(core_axis_name="c", subcore_axis_name="s"),
        scratch_types=[
            pltpu.VMEM_SHARED((N_NODES, D_FEAT), jnp.float32),
            pltpu.VMEM_SHARED((N_NODES, D_DEG), jnp.float32),
            pltpu.VMEM((NUM_CHUNKS, CHUNK), jnp.int32),
            pltpu.VMEM((NUM_CHUNKS, CHUNK), jnp.int32),
            pltpu.VMEM((CHUNK, D_FEAT), jnp.float32),
            pltpu.VMEM((CHUNK, D_FEAT), jnp.float32),
            pltpu.VMEM((CHUNK, D_DEG), jnp.float32),
            pltpu.SemaphoreType.DMA,
            pltpu.SemaphoreType.DMA,
            pltpu.SemaphoreType.DMA,
            pltpu.SemaphoreType.DMA,
        ],
        compiler_params=pltpu.CompilerParams(use_tc_tiling_on_sc=False),
    )(_sc_body)


def _tc_body(pf_ref, pd_ref, f_ref, wt_ref, b_ref, o_ref):
    feat_sum = pf_ref[0] + pf_ref[1]
    deg = pd_ref[0] + pd_ref[1]
    mean = feat_sum / jnp.maximum(deg, 1.0)
    h = jnp.where(deg > 0.0, mean, f_ref[...])
    y = jnp.dot(h, wt_ref[...], preferred_element_type=jnp.float32)
    o_ref[...] = jnp.maximum(y + b_ref[...], 0.0)


_BR = 1000


def _tc_finish(pfeat, pdeg, features, wt, b2):
    grid = (N_NODES // _BR,)
    return pl.pallas_call(
        _tc_body,
        grid=grid,
        in_specs=[
            pl.BlockSpec((NUM_CORES, _BR, D_FEAT), lambda i: (0, i, 0)),
            pl.BlockSpec((NUM_CORES, _BR, 1), lambda i: (0, i, 0)),
            pl.BlockSpec((_BR, D_FEAT), lambda i: (i, 0)),
            pl.BlockSpec((D_FEAT, D_FEAT), lambda i: (0, 0)),
            pl.BlockSpec((1, D_FEAT), lambda i: (0, 0)),
        ],
        out_specs=pl.BlockSpec((_BR, D_FEAT), lambda i: (i, 0)),
        out_shape=jax.ShapeDtypeStruct((N_NODES, D_FEAT), jnp.float32),
    )(pfeat, pdeg, features, wt, b2)


def kernel(features, edge_index, W, b):
    src = edge_index[0].astype(jnp.int32).reshape(NUM_WORKERS, NUM_CHUNKS, CHUNK)
    dst = edge_index[1].astype(jnp.int32).reshape(NUM_WORKERS, NUM_CHUNKS, CHUNK)
    zf = jnp.zeros((ROWS_PER_TILE, D_FEAT), jnp.float32)
    zd = jnp.zeros((ROWS_PER_TILE, D_DEG), jnp.float32)
    pfeat, pdeg = _sc_agg()(features, src, dst, zf, zd)
    pfeat = pfeat.reshape(NUM_CORES, N_NODES, D_FEAT)
    pdeg = pdeg.reshape(NUM_CORES, N_NODES, D_DEG)[:, :, :1]
    return _tc_finish(pfeat, pdeg, features, W.T, b.reshape(1, D_FEAT))


# speedup vs baseline: 1.3224x; 1.2344x over previous
import functools

import jax
import jax.numpy as jnp
from jax import lax
from jax.experimental import pallas as pl
from jax.experimental.pallas import tpu as pltpu
from jax.experimental.pallas import tpu_sc as plsc

N_NODES = 10000
N_EDGES = 320000
D_FEAT = 128
D_DEG = 16

NUM_CORES = 2
NUM_SUBCORES = 16
NUM_WORKERS = NUM_CORES * NUM_SUBCORES
EDGES_PER_WORKER = N_EDGES // NUM_WORKERS
CHUNK = 80
NUM_CHUNKS = EDGES_PER_WORKER // CHUNK
ROWS_PER_TILE = N_NODES // NUM_SUBCORES


def _sc_body(feat_hbm, src_hbm, dst_hbm, zf_hbm, zd_hbm, outf_hbm, outd_hbm,
             accf_sh, accd_sh, src_v, dst_v, rows0, rows1, ones_v,
             g0, g1, sd, zs):
    cid = lax.axis_index("c")
    sid = lax.axis_index("s")
    wid = cid * NUM_SUBCORES + sid
    row0 = sid * ROWS_PER_TILE

    pltpu.async_copy(zf_hbm, accf_sh.at[pl.ds(row0, ROWS_PER_TILE)], zs)
    pltpu.async_copy(zd_hbm, accd_sh.at[pl.ds(row0, ROWS_PER_TILE)], zs)

    ovec = jnp.ones((2, 16), jnp.bfloat16)

    def _orow(i, _):
        ones_v[pl.ds(2 * i, 2), :] = ovec
        return 0

    lax.fori_loop(0, CHUNK // 2, _orow, 0)

    pltpu.sync_copy(src_hbm.at[wid], src_v)
    pltpu.sync_copy(dst_hbm.at[wid], dst_v)

    pltpu.make_async_copy(zf_hbm, accf_sh.at[pl.ds(row0, ROWS_PER_TILE)], zs).wait()
    pltpu.make_async_copy(zd_hbm, accd_sh.at[pl.ds(row0, ROWS_PER_TILE)], zs).wait()

    plsc.subcore_barrier()

    pltpu.async_copy(feat_hbm.at[src_v.at[0]], rows0, g0)
    pltpu.async_copy(feat_hbm.at[src_v.at[1]], rows1, g1)

    def _step(i, _):
        j = 2 * i
        pltpu.make_async_copy(feat_hbm.at[src_v.at[j]], rows0, g0).wait()

        @pl.when(i > 0)
        def _():
            pltpu.make_async_copy(ones_v, accd_sh.at[dst_v.at[0]], sd).wait()

        pltpu.async_copy(ones_v, accd_sh.at[dst_v.at[j]], sd, add=True)
        pltpu.sync_copy(rows0, accf_sh.at[dst_v.at[j]], add=True)

        @pl.when(i < NUM_CHUNKS // 2 - 1)
        def _():
            pltpu.async_copy(feat_hbm.at[src_v.at[j + 2]], rows0, g0)

        pltpu.make_async_copy(feat_hbm.at[src_v.at[j + 1]], rows1, g1).wait()
        pltpu.make_async_copy(ones_v, accd_sh.at[dst_v.at[0]], sd).wait()
        pltpu.async_copy(ones_v, accd_sh.at[dst_v.at[j + 1]], sd, add=True)
        pltpu.sync_copy(rows1, accf_sh.at[dst_v.at[j + 1]], add=True)

        @pl.when(i < NUM_CHUNKS // 2 - 1)
        def _():
            pltpu.async_copy(feat_hbm.at[src_v.at[j + 3]], rows1, g1)

        return 0

    lax.fori_loop(0, NUM_CHUNKS // 2, _step, 0)

    if NUM_CHUNKS % 2:
        last = NUM_CHUNKS - 1
        pltpu.async_copy(feat_hbm.at[src_v.at[last]], rows0, g0)
        pltpu.make_async_copy(feat_hbm.at[src_v.at[last]], rows0, g0).wait()
        pltpu.make_async_copy(ones_v, accd_sh.at[dst_v.at[0]], sd).wait()
        pltpu.async_copy(ones_v, accd_sh.at[dst_v.at[last]], sd, add=True)
        pltpu.sync_copy(rows0, accf_sh.at[dst_v.at[last]], add=True)

    pltpu.make_async_copy(ones_v, accd_sh.at[dst_v.at[0]], sd).wait()

    plsc.subcore_barrier()

    out_base = cid * N_NODES + sid * ROWS_PER_TILE
    pltpu.sync_copy(accf_sh.at[pl.ds(row0, ROWS_PER_TILE)],
                    outf_hbm.at[pl.ds(out_base, ROWS_PER_TILE)])
    pltpu.sync_copy(accd_sh.at[pl.ds(row0, ROWS_PER_TILE)],
                    outd_hbm.at[pl.ds(out_base, ROWS_PER_TILE)])


@functools.lru_cache(maxsize=1)
def _sc_agg():
    return functools.partial(
        pl.kernel,
        out_type=(
            jax.ShapeDtypeStruct((NUM_CORES * N_NODES, D_FEAT), jnp.float32),
            jax.ShapeDtypeStruct((NUM_CORES * N_NODES, D_DEG), jnp.bfloat16),
        ),
        mesh=plsc.VectorSubcoreMesh(core_axis_name="c", subcore_axis_name="s"),
        scratch_types=[
            pltpu.VMEM_SHARED((N_NODES, D_FEAT), jnp.float32),
            pltpu.VMEM_SHARED((N_NODES, D_DEG), jnp.bfloat16),
            pltpu.VMEM((NUM_CHUNKS, CHUNK), jnp.int32),
            pltpu.VMEM((NUM_CHUNKS, CHUNK), jnp.int32),
            pltpu.VMEM((CHUNK, D_FEAT), jnp.float32),
            pltpu.VMEM((CHUNK, D_FEAT), jnp.float32),
            pltpu.VMEM((CHUNK, D_DEG), jnp.bfloat16),
            pltpu.SemaphoreType.DMA,
            pltpu.SemaphoreType.DMA,
            pltpu.SemaphoreType.DMA,
            pltpu.SemaphoreType.DMA,
        ],
        compiler_params=pltpu.CompilerParams(use_tc_tiling_on_sc=False),
    )(_sc_body)


def _tc_body(pf_ref, pd_ref, f_ref, wt_ref, b_ref, o_ref):
    feat_sum = pf_ref[0] + pf_ref[1]
    deg = pd_ref[0].astype(jnp.float32) + pd_ref[1].astype(jnp.float32)
    mean = feat_sum / jnp.maximum(deg, 1.0)
    h = jnp.where(deg > 0.0, mean, f_ref[...])
    y = jnp.dot(h, wt_ref[...], preferred_element_type=jnp.float32)
    o_ref[...] = jnp.maximum(y + b_ref[...], 0.0)


_BR = 1000


def _tc_finish(pfeat, pdeg, features, wt, b2):
    grid = (N_NODES // _BR,)
    return pl.pallas_call(
        _tc_body,
        grid=grid,
        in_specs=[
            pl.BlockSpec((NUM_CORES, _BR, D_FEAT), lambda i: (0, i, 0)),
            pl.BlockSpec((NUM_CORES, _BR, 1), lambda i: (0, i, 0)),
            pl.BlockSpec((_BR, D_FEAT), lambda i: (i, 0)),
            pl.BlockSpec((D_FEAT, D_FEAT), lambda i: (0, 0)),
            pl.BlockSpec((1, D_FEAT), lambda i: (0, 0)),
        ],
        out_specs=pl.BlockSpec((_BR, D_FEAT), lambda i: (i, 0)),
        out_shape=jax.ShapeDtypeStruct((N_NODES, D_FEAT), jnp.float32),
    )(pfeat, pdeg, features, wt, b2)


def kernel(features, edge_index, W, b):
    src = edge_index[0].astype(jnp.int32).reshape(NUM_WORKERS, NUM_CHUNKS, CHUNK)
    dst = edge_index[1].astype(jnp.int32).reshape(NUM_WORKERS, NUM_CHUNKS, CHUNK)
    zf = jnp.zeros((ROWS_PER_TILE, D_FEAT), jnp.float32)
    zd = jnp.zeros((ROWS_PER_TILE, D_DEG), jnp.bfloat16)
    pfeat, pdeg = _sc_agg()(features, src, dst, zf, zd)
    pfeat = pfeat.reshape(NUM_CORES, N_NODES, D_FEAT)
    pdeg = pdeg.reshape(NUM_CORES, N_NODES, D_DEG)[:, :, :1]
    return _tc_finish(pfeat, pdeg, features, W.T, b.reshape(1, D_FEAT))
